# baseline (speedup 1.0000x reference)
import functools

import jax
import jax.numpy as jnp
from jax.experimental import pallas as pl

_HIGH = jax.lax.Precision.HIGHEST
_EPS_BN = 1e-5
_EPS_D = 1e-8

MBLK = 512


def _prep_body(feat_ref, w1_ref, g1_ref, b1_ref, tf_ref, w2_ref, g2_ref,
               b2_ref, f_ref, sc2_ref, sh2_ref):
    B = feat_ref.shape[0]
    N = feat_ref.shape[2]
    Mtot = tf_ref.shape[2]
    w1 = w1_ref[...]
    ys = [jnp.dot(w1, feat_ref[b], preferred_element_type=jnp.float32,
                  precision=_HIGH) for b in range(B)]
    cnt1 = float(B * N)
    mean1 = sum(jnp.sum(y, axis=1, keepdims=True) for y in ys) / cnt1
    var1 = sum(jnp.sum((y - mean1) ** 2, axis=1, keepdims=True)
               for y in ys) / cnt1
    sc1 = g1_ref[...] * jax.lax.rsqrt(var1 + _EPS_BN)
    sh1 = b1_ref[...] - mean1 * sc1
    for b in range(B):
        f_ref[b] = jnp.maximum(ys[b] * sc1 + sh1, 0.0)

    cnt2 = float(B * Mtot)
    w2 = w2_ref[...]
    xmean = sum(jnp.sum(tf_ref[b], axis=1, keepdims=True)
                for b in range(B)) / cnt2
    smom = sum(jax.lax.dot_general(tf_ref[b], tf_ref[b],
                                   (((1,), (1,)), ((), ())),
                                   preferred_element_type=jnp.float32,
                                   precision=_HIGH) for b in range(B))
    mu2 = jnp.dot(w2, xmean, preferred_element_type=jnp.float32,
                  precision=_HIGH)
    ey2 = jnp.sum(jnp.dot(w2, smom, preferred_element_type=jnp.float32,
                          precision=_HIGH) * w2, axis=1,
                  keepdims=True) / cnt2
    var2 = ey2 - mu2 * mu2
    sc2 = g2_ref[...] * jax.lax.rsqrt(var2 + _EPS_BN)
    sc2_ref[...] = sc2
    sh2_ref[...] = b2_ref[...] - mu2 * sc2


def _main_body(xyz_ref, txyz_ref, f_ref, tf_ref, w2_ref, sc2_ref, sh2_ref,
               o_ref):
    S = xyz_ref[0]
    T = txyz_ref[0]
    N = S.shape[0]
    MB = T.shape[1]
    st = jnp.dot(S, T, preferred_element_type=jnp.float32, precision=_HIGH)
    s2 = jnp.sum(S * S, axis=1, keepdims=True)
    t2 = jnp.sum(T * T, axis=0, keepdims=True)
    d2 = jnp.maximum((s2 + t2) - (st + st), 0.0)
    iota0 = jax.lax.broadcasted_iota(jnp.int32, (N, MB), 0)
    key = (jax.lax.bitcast_convert_type(d2, jnp.int32) &
           jnp.int32(~2047)) | iota0
    kmax = jnp.int32(0x7FFFFFFF)
    k0 = jnp.min(key, axis=0, keepdims=True)
    m1 = jnp.where(key == k0, kmax, key)
    k1 = jnp.min(m1, axis=0, keepdims=True)
    m2 = jnp.where(m1 == k1, kmax, m1)
    k2 = jnp.min(m2, axis=0, keepdims=True)
    recips = []
    for kk in (k0, k1, k2):
        dq = jax.lax.bitcast_convert_type(kk & jnp.int32(~2047), jnp.float32)
        recips.append(1.0 / (jnp.sqrt(dq) + _EPS_D))
    norm = recips[0] + recips[1] + recips[2]
    wmat = jnp.where(key == k0, recips[0] / norm,
                     jnp.where(key == k1, recips[1] / norm,
                               jnp.where(key == k2, recips[2] / norm, 0.0)))
    interp = jnp.dot(f_ref[0], wmat, preferred_element_type=jnp.float32,
                     precision=_HIGH)
    y2 = jnp.dot(w2_ref[...], tf_ref[0], preferred_element_type=jnp.float32,
                 precision=_HIGH)
    t = jnp.maximum(y2 * sc2_ref[...] + sh2_ref[...], 0.0)
    o_ref[0] = t + interp


@jax.jit
def kernel(xyz, feature, target_xyz, target_feature, W1, gamma1, beta1, W2,
           gamma2, beta2):
    B, N, _ = xyz.shape
    M = target_xyz.shape[1]
    C2 = W1.shape[0]
    txyz_t = jnp.transpose(target_xyz, (0, 2, 1))
    g1 = gamma1.reshape(C2, 1)
    b1 = beta1.reshape(C2, 1)
    g2 = gamma2.reshape(C2, 1)
    b2 = beta2.reshape(C2, 1)

    f, sc2, sh2 = pl.pallas_call(
        _prep_body,
        out_shape=[
            jax.ShapeDtypeStruct((B, C2, N), jnp.float32),
            jax.ShapeDtypeStruct((C2, 1), jnp.float32),
            jax.ShapeDtypeStruct((C2, 1), jnp.float32),
        ],
    )(feature, W1, g1, b1, target_feature, W2, g2, b2)

    grid = (B, M // MBLK)
    out = pl.pallas_call(
        _main_body,
        grid=grid,
        in_specs=[
            pl.BlockSpec((1, N, 3), lambda b, j: (b, 0, 0)),
            pl.BlockSpec((1, 3, MBLK), lambda b, j: (b, 0, j)),
            pl.BlockSpec((1, C2, N), lambda b, j: (b, 0, 0)),
            pl.BlockSpec((1, C2, MBLK), lambda b, j: (b, 0, j)),
            pl.BlockSpec((C2, C2), lambda b, j: (0, 0)),
            pl.BlockSpec((C2, 1), lambda b, j: (0, 0)),
            pl.BlockSpec((C2, 1), lambda b, j: (0, 0)),
        ],
        out_specs=pl.BlockSpec((1, C2, MBLK), lambda b, j: (b, 0, j)),
        out_shape=jax.ShapeDtypeStruct((B, C2, M), jnp.float32),
    )(xyz, txyz_t, f, target_feature, W2, sc2, sh2)
    return out


# speedup vs baseline: 1.4314x; 1.4314x over previous
import functools

import jax
import jax.numpy as jnp
from jax.experimental import pallas as pl

_HIGH = jax.lax.Precision.HIGHEST
_EPS_BN = 1e-5
_EPS_D = 1e-8

MBLK = 512


def _prep_body(feat_ref, w1_ref, g1_ref, b1_ref, tf_ref, w2_ref, g2_ref,
               b2_ref, f_ref, sc2_ref, sh2_ref):
    B = feat_ref.shape[0]
    N = feat_ref.shape[2]
    Mtot = tf_ref.shape[2]
    w1 = w1_ref[...]
    ys = [jnp.dot(w1, feat_ref[b], preferred_element_type=jnp.float32,
                  precision=_HIGH) for b in range(B)]
    cnt1 = float(B * N)
    mean1 = sum(jnp.sum(y, axis=1, keepdims=True) for y in ys) / cnt1
    var1 = sum(jnp.sum((y - mean1) ** 2, axis=1, keepdims=True)
               for y in ys) / cnt1
    sc1 = g1_ref[...] * jax.lax.rsqrt(var1 + _EPS_BN)
    sh1 = b1_ref[...] - mean1 * sc1
    for b in range(B):
        f_ref[b] = jnp.maximum(ys[b] * sc1 + sh1, 0.0)

    cnt2 = float(B * Mtot)
    w2 = w2_ref[...]
    xmean = sum(jnp.sum(tf_ref[b], axis=1, keepdims=True)
                for b in range(B)) / cnt2
    smom = sum(jax.lax.dot_general(tf_ref[b], tf_ref[b],
                                   (((1,), (1,)), ((), ())),
                                   preferred_element_type=jnp.float32,
                                   precision=_HIGH) for b in range(B))
    mu2 = jnp.dot(w2, xmean, preferred_element_type=jnp.float32,
                  precision=_HIGH)
    ey2 = jnp.sum(jnp.dot(w2, smom, preferred_element_type=jnp.float32,
                          precision=_HIGH) * w2, axis=1,
                  keepdims=True) / cnt2
    var2 = ey2 - mu2 * mu2
    sc2 = g2_ref[...] * jax.lax.rsqrt(var2 + _EPS_BN)
    sc2_ref[...] = sc2
    sh2_ref[...] = b2_ref[...] - mu2 * sc2


def _main_body(xyz_ref, txyz_ref, f_ref, tf_ref, w2_ref, sc2_ref, sh2_ref,
               o_ref):
    S = xyz_ref[0]
    T = txyz_ref[0]
    N = S.shape[0]
    MB = T.shape[1]
    d2 = None
    for c in range(3):
        diff = S[:, c:c + 1] - T[c:c + 1, :]
        d2 = diff * diff if d2 is None else d2 + diff * diff
    iota0 = jax.lax.broadcasted_iota(jnp.int32, (N, MB), 0)
    key = jax.lax.bitcast_convert_type(
        (jax.lax.bitcast_convert_type(d2, jnp.int32) & jnp.int32(~2047))
        | iota0, jnp.float32)
    kmax = jnp.float32(jnp.inf)
    k0 = jnp.min(key, axis=0, keepdims=True)
    m1 = jnp.where(key == k0, kmax, key)
    k1 = jnp.min(m1, axis=0, keepdims=True)
    m2 = jnp.where(m1 == k1, kmax, m1)
    k2 = jnp.min(m2, axis=0, keepdims=True)
    recips = []
    for kk in (k0, k1, k2):
        dq = jax.lax.bitcast_convert_type(
            jax.lax.bitcast_convert_type(kk, jnp.int32) & jnp.int32(~2047),
            jnp.float32)
        recips.append(1.0 / (jnp.sqrt(dq) + _EPS_D))
    norm = recips[0] + recips[1] + recips[2]
    wmat = jnp.where(key == k0, recips[0] / norm,
                     jnp.where(key == k1, recips[1] / norm,
                               jnp.where(key == k2, recips[2] / norm, 0.0)))
    interp = jnp.dot(f_ref[0], wmat, preferred_element_type=jnp.float32,
                     precision=_HIGH)
    y2 = jnp.dot(w2_ref[...], tf_ref[0], preferred_element_type=jnp.float32,
                 precision=_HIGH)
    t = jnp.maximum(y2 * sc2_ref[...] + sh2_ref[...], 0.0)
    o_ref[0] = t + interp


@jax.jit
def kernel(xyz, feature, target_xyz, target_feature, W1, gamma1, beta1, W2,
           gamma2, beta2):
    B, N, _ = xyz.shape
    M = target_xyz.shape[1]
    C2 = W1.shape[0]
    txyz_t = jnp.transpose(target_xyz, (0, 2, 1))
    g1 = gamma1.reshape(C2, 1)
    b1 = beta1.reshape(C2, 1)
    g2 = gamma2.reshape(C2, 1)
    b2 = beta2.reshape(C2, 1)

    f, sc2, sh2 = pl.pallas_call(
        _prep_body,
        out_shape=[
            jax.ShapeDtypeStruct((B, C2, N), jnp.float32),
            jax.ShapeDtypeStruct((C2, 1), jnp.float32),
            jax.ShapeDtypeStruct((C2, 1), jnp.float32),
        ],
    )(feature, W1, g1, b1, target_feature, W2, g2, b2)

    grid = (B, M // MBLK)
    out = pl.pallas_call(
        _main_body,
        grid=grid,
        in_specs=[
            pl.BlockSpec((1, N, 3), lambda b, j: (b, 0, 0)),
            pl.BlockSpec((1, 3, MBLK), lambda b, j: (b, 0, j)),
            pl.BlockSpec((1, C2, N), lambda b, j: (b, 0, 0)),
            pl.BlockSpec((1, C2, MBLK), lambda b, j: (b, 0, j)),
            pl.BlockSpec((C2, C2), lambda b, j: (0, 0)),
            pl.BlockSpec((C2, 1), lambda b, j: (0, 0)),
            pl.BlockSpec((C2, 1), lambda b, j: (0, 0)),
        ],
        out_specs=pl.BlockSpec((1, C2, MBLK), lambda b, j: (b, 0, j)),
        out_shape=jax.ShapeDtypeStruct((B, C2, M), jnp.float32),
    )(xyz, txyz_t, f, target_feature, W2, sc2, sh2)
    return out
